# baseline (speedup 1.0000x reference)
import functools

import jax
import jax.numpy as jnp
from jax import lax
from jax.experimental import pallas as pl
from jax.experimental.pallas import tpu as pltpu
from jax.experimental.pallas import tpu_sc as plsc

N_NODES = 10000
N_EDGES = 320000
D_FEAT = 128
N_HIDDEN = 96
HP = 128

_NC = 2
_NS = 16
_NW = _NC * _NS
_EPT = N_EDGES // _NW
_CH = 200
_NCHUNK = _EPT // _CH
_CHS = 80
_NCHUNKS = _EPT // _CHS
_ZB = 1000

_EDGE_BLK = 3200
_NODE_BLK = 2000


def _sc_mesh():
    return plsc.VectorSubcoreMesh(
        core_axis_name="c", subcore_axis_name="s",
        num_cores=_NC, num_subcores=_NS)


def _pad_cols(w):
    pad = [(0, 0)] * (w.ndim - 1) + [(0, HP - w.shape[-1])]
    return jnp.pad(w, pad)


def _gather_body(ept, a_hbm, b_hbm, src_hbm, dst_hbm, e_hbm,
                 idx_s, idx_d, rows, sem_a, sem_b, sem_o):
    nchunk = ept // _CH
    c = lax.axis_index("c")
    s = lax.axis_index("s")
    wid = s * _NC + c
    base0 = wid * ept

    pltpu.sync_copy(src_hbm.at[pl.ds(base0, ept)], idx_s)
    pltpu.sync_copy(dst_hbm.at[pl.ds(base0, ept)], idx_d)

    def start_a(k, p):
        @pl.when(k >= 2)
        def _():
            pltpu.make_async_copy(
                rows.at[p], e_hbm.at[pl.ds(base0 + (k - 2) * _CH, _CH)],
                sem_o.at[p]).wait()
        pltpu.async_copy(a_hbm.at[idx_s.at[pl.ds(k * _CH, _CH)]],
                         rows.at[p], sem_a.at[p])

    def start_b(k, p):
        pltpu.make_async_copy(a_hbm.at[idx_s.at[pl.ds(k * _CH, _CH)]],
                              rows.at[p], sem_a.at[p]).wait()
        pltpu.async_copy(b_hbm.at[idx_d.at[pl.ds(k * _CH, _CH)]],
                         rows.at[p], sem_b.at[p], add=True)

    def store(k, p):
        pltpu.make_async_copy(b_hbm.at[idx_d.at[pl.ds(k * _CH, _CH)]],
                              rows.at[p], sem_b.at[p]).wait()
        pltpu.async_copy(rows.at[p], e_hbm.at[pl.ds(base0 + k * _CH, _CH)],
                         sem_o.at[p])

    start_a(0, 0)

    def pair(i, carry):
        k0 = 2 * i
        k1 = k0 + 1
        start_a(k1, 1)
        start_b(k0, 0)
        store(k0, 0)

        @pl.when(k0 + 2 < nchunk)
        def _():
            start_a(k0 + 2, 0)
        start_b(k1, 1)
        store(k1, 1)
        return carry

    lax.fori_loop(0, nchunk // 2, pair, 0)

    if nchunk % 2:
        start_b(nchunk - 1, 0)
        store(nchunk - 1, 0)
        pltpu.make_async_copy(
            rows.at[1], e_hbm.at[pl.ds(base0 + (nchunk - 2) * _CH, _CH)],
            sem_o.at[1]).wait()
        pltpu.make_async_copy(
            rows.at[0], e_hbm.at[pl.ds(base0 + (nchunk - 1) * _CH, _CH)],
            sem_o.at[0]).wait()
    else:
        pltpu.make_async_copy(
            rows.at[0], e_hbm.at[pl.ds(base0 + (nchunk - 2) * _CH, _CH)],
            sem_o.at[0]).wait()
        pltpu.make_async_copy(
            rows.at[1], e_hbm.at[pl.ds(base0 + (nchunk - 1) * _CH, _CH)],
            sem_o.at[1]).wait()


@functools.partial(jax.jit, static_argnums=(4,))
def _sc_gather(a, b, src, dst, nedges):
    ept = nedges // _NW
    f = pl.kernel(
        functools.partial(_gather_body, ept),
        out_type=jax.ShapeDtypeStruct((nedges, HP), jnp.float32),
        mesh=_sc_mesh(),
        scratch_types=[
            pltpu.VMEM((ept,), jnp.int32),
            pltpu.VMEM((ept,), jnp.int32),
            pltpu.VMEM((2, _CH, HP), jnp.float32),
            pltpu.SemaphoreType.DMA((2,)),
            pltpu.SemaphoreType.DMA((2,)),
            pltpu.SemaphoreType.DMA((2,)),
        ],
    )
    return f(a, b, src, dst)


def _scatter_body(ept, chs, m_hbm, dst_hbm, zeros_hbm, out_hbm,
                  idx, mbuf, acc, sem_m):
    nchunks = ept // chs
    c = lax.axis_index("c")
    s = lax.axis_index("s")
    wid = s * _NC + c
    base0 = wid * ept

    @pl.when(s < 10)
    def _zero():
        pltpu.sync_copy(zeros_hbm.at[pl.ds(s * _ZB, _ZB)],
                        acc.at[pl.ds(s * _ZB, _ZB)])
    pltpu.sync_copy(dst_hbm.at[pl.ds(base0, ept)], idx)
    plsc.subcore_barrier()

    def load(k, p):
        pltpu.async_copy(m_hbm.at[pl.ds(base0 + k * chs, chs)],
                         mbuf.at[p], sem_m.at[p])

    def add(k, p):
        pltpu.make_async_copy(m_hbm.at[pl.ds(base0 + k * chs, chs)],
                              mbuf.at[p], sem_m.at[p]).wait()
        pltpu.sync_copy(mbuf.at[p], acc.at[idx.at[pl.ds(k * chs, chs)]],
                        add=True)

    load(0, 0)

    def pair(i, carry):
        k0 = 2 * i
        k1 = k0 + 1
        load(k1, 1)
        add(k0, 0)

        @pl.when(k0 + 2 < nchunks)
        def _():
            load(k0 + 2, 0)
        add(k1, 1)
        return carry

    lax.fori_loop(0, nchunks // 2, pair, 0)
    if nchunks % 2:
        add(nchunks - 1, 0)
    plsc.subcore_barrier()

    @pl.when(s < 10)
    def _writeback():
        pltpu.sync_copy(acc.at[pl.ds(s * _ZB, _ZB)],
                        out_hbm.at[pl.ds(c * N_NODES + s * _ZB, _ZB)])


@functools.partial(jax.jit, static_argnums=(3, 4))
def _sc_scatter(m, dst3, zeros, nedges, chs):
    ept = nedges // _NW
    nchunks = ept // chs
    f = pl.kernel(
        functools.partial(_scatter_body, ept, chs),
        out_type=jax.ShapeDtypeStruct((2 * N_NODES, HP), jnp.float32),
        mesh=_sc_mesh(),
        scratch_types=[
            pltpu.VMEM((ept,), jnp.int32),
            pltpu.VMEM((2, chs, HP), jnp.float32),
            pltpu.VMEM_SHARED((N_NODES, HP), jnp.float32),
            pltpu.SemaphoreType.DMA((2,)),
        ],
    )
    return f(m, dst3, zeros)


def _dot(x, w):
    return jax.lax.dot_general(x, w, (((1,), (0,)), ((), ())),
                               preferred_element_type=jnp.float32)


def _pre_body(x_ref, w0, b0, w1, b1, w2, b2, w3, b3, wa, ba, wb,
              h_ref, a_ref, bo_ref):
    h = jnp.maximum(_dot(x_ref[...], w0[...]) + b0[...], 0.0)
    h = jnp.maximum(_dot(h, w1[...]) + b1[...], 0.0)
    h = jnp.maximum(_dot(h, w2[...]) + b2[...], 0.0)
    h = _dot(h, w3[...]) + b3[...]
    h_ref[...] = h
    a_ref[...] = _dot(h, wa[...]) + ba[...]
    bo_ref[...] = _dot(h, wb[...])


@jax.jit
def _tc_pre(x, pre, wa, ba, wb):
    nblk = N_NODES // _NODE_BLK
    row = lambda i: (i, 0)
    cst = lambda i: (0, 0)
    ws = []
    specs = [pl.BlockSpec((_NODE_BLK, D_FEAT), row)]
    for (w, b) in pre:
        ws += [w, b.reshape(1, -1)]
        specs += [pl.BlockSpec(w.shape, cst), pl.BlockSpec((1, w.shape[1]), cst)]
    ws += [wa, ba.reshape(1, -1), wb]
    specs += [pl.BlockSpec(wa.shape, cst), pl.BlockSpec((1, HP), cst),
              pl.BlockSpec(wb.shape, cst)]
    return pl.pallas_call(
        _pre_body,
        grid=(nblk,),
        in_specs=specs,
        out_specs=[pl.BlockSpec((_NODE_BLK, N_HIDDEN), row),
                   pl.BlockSpec((_NODE_BLK, HP), row),
                   pl.BlockSpec((_NODE_BLK, HP), row)],
        out_shape=[jax.ShapeDtypeStruct((N_NODES, N_HIDDEN), jnp.float32),
                   jax.ShapeDtypeStruct((N_NODES, HP), jnp.float32),
                   jax.ShapeDtypeStruct((N_NODES, HP), jnp.float32)],
    )(x, *ws)


def _edge_body(e_ref, w2, b2, w3, b3, w4, b4, m_ref):
    e = jnp.maximum(e_ref[...], 0.0)
    e = jnp.maximum(_dot(e, w2[...]) + b2[...], 0.0)
    e = jnp.maximum(_dot(e, w3[...]) + b3[...], 0.0)
    m_ref[...] = _dot(e, w4[...]) + b4[...]


@jax.jit
def _tc_edge(e1, w2p, b2, w3, b3, w4p, b4p):
    nblk = N_EDGES // _EDGE_BLK
    row = lambda i: (i, 0)
    cst = lambda i: (0, 0)
    ws = [w2p, b2.reshape(1, -1), w3, b3.reshape(1, -1),
          w4p, b4p.reshape(1, -1)]
    specs = [pl.BlockSpec((_EDGE_BLK, HP), row)]
    for w in ws:
        specs.append(pl.BlockSpec(w.shape, cst))
    return pl.pallas_call(
        _edge_body,
        grid=(nblk,),
        in_specs=specs,
        out_specs=pl.BlockSpec((_EDGE_BLK, HP), row),
        out_shape=jax.ShapeDtypeStruct((N_EDGES, HP), jnp.float32),
    )(e1, *ws)


def _node_call(h, pa, pb, node, heads, out_dims, final):
    p0, p1 = pa[:N_NODES], pa[N_NODES:]
    p2, p3 = pb[:N_NODES], pb[N_NODES:]
    nblk = N_NODES // _NODE_BLK
    row = lambda i: (i, 0)
    cst = lambda i: (0, 0)
    (wn1, bn1), n2, n3, n4 = node
    wna = wn1[:N_HIDDEN]
    wnb = jnp.pad(wn1[N_HIDDEN:], ((0, HP - N_HIDDEN), (0, 0)))
    ws = [wna, wnb, bn1.reshape(1, -1)]
    specs = [pl.BlockSpec((_NODE_BLK, N_HIDDEN), row)]
    specs += [pl.BlockSpec((_NODE_BLK, HP), row)] * 4
    specs += [pl.BlockSpec(wna.shape, cst), pl.BlockSpec(wnb.shape, cst),
              pl.BlockSpec((1, N_HIDDEN), cst)]
    for (w, b) in (n2, n3, n4):
        ws += [w, b.reshape(1, -1)]
        specs += [pl.BlockSpec(w.shape, cst), pl.BlockSpec((1, w.shape[1]), cst)]
    for hmat in heads:
        shp = hmat.shape if hmat.ndim == 2 else (1, hmat.shape[-1])
        ws.append(hmat.reshape(shp))
        specs.append(pl.BlockSpec(shp, cst))

    def body(h_ref, p0_ref, p1_ref, p2_ref, p3_ref, rwna, rwnb, rbn1,
             rw2, rb2, rw3, rb3, rw4, rb4, *rest):
        agg = ((p0_ref[...] + p1_ref[...])
               + (p2_ref[...] + p3_ref[...]))
        e = jnp.maximum(_dot(h_ref[...], rwna[...]) + _dot(agg, rwnb[...])
                        + rbn1[...], 0.0)
        e = jnp.maximum(_dot(e, rw2[...]) + rb2[...], 0.0)
        e = jnp.maximum(_dot(e, rw3[...]) + rb3[...], 0.0)
        hn = _dot(e, rw4[...]) + rb4[...]
        if final:
            rwo, rbo, out_ref = rest
            out_ref[...] = _dot(hn, rwo[...]) + rbo[...]
        else:
            rwa, rba, rwb, h_out, a_out, b_out = rest
            h_out[...] = hn
            a_out[...] = _dot(hn, rwa[...]) + rba[...]
            b_out[...] = _dot(hn, rwb[...])

    return pl.pallas_call(
        body,
        grid=(nblk,),
        in_specs=specs,
        out_specs=[pl.BlockSpec((_NODE_BLK, d), row) for d, _ in out_dims],
        out_shape=[jax.ShapeDtypeStruct((N_NODES, d), t)
                   for d, t in out_dims],
    )(h, p0, p1, p2, p3, *ws)


@jax.jit
def _tc_node_mid(h, pa, pb, node, wa, ba, wb):
    return _node_call(h, pa, pb, node, [wa, ba, wb],
                      [(N_HIDDEN, jnp.float32), (HP, jnp.float32),
                       (HP, jnp.float32)], final=False)


@jax.jit
def _tc_node_final(h, pa, pb, node, wo, bo):
    return _node_call(h, pa, pb, node, [wo, bo], [(D_FEAT, jnp.float32)],
                      final=True)


def kernel(x, params, edge_index):
    src = edge_index[0].astype(jnp.int32)
    dst = edge_index[1].astype(jnp.int32)
    msg = params['msg']
    wm1, bm1 = msg[0]
    wma = _pad_cols(wm1[:N_HIDDEN])
    wmb = _pad_cols(wm1[N_HIDDEN:])
    bm1p = _pad_cols(bm1)
    (w2, b2), (w3, b3), (w4, b4) = msg[1], msg[2], msg[3]
    w2p = jnp.pad(w2, ((0, HP - N_HIDDEN), (0, 0)))
    w4p = _pad_cols(w4)
    b4p = _pad_cols(b4)
    wo, bo = params['out']

    h, a, b = _tc_pre(x, params['pre'], wma, bm1p, wmb)
    zeros = jnp.zeros((N_NODES, HP), jnp.float32)

    half = N_EDGES // 2
    src_a, src_b = src[:half], src[half:]
    dst_a, dst_b = dst[:half], dst[half:]

    out = None
    for step in range(2):
        e1a = _sc_gather(a, b, src_a, dst_a, half)
        e1b = _sc_gather(a, b, src_b, dst_b, half)
        ma = _tc_edge(e1a, w2p, b2, w3, b3, w4p, b4p)
        mb = _tc_edge(e1b, w2p, b2, w3, b3, w4p, b4p)
        pa = _sc_scatter(ma, dst_a, zeros, half, 40)
        pb = _sc_scatter(mb, dst_b, zeros, half, 40)
        if step == 0:
            h, a, b = _tc_node_mid(h, pa, pb, params['node'], wma, bm1p, wmb)
        else:
            (out,) = _tc_node_final(h, pa, pb, params['node'], wo, bo)
    return out


# speedup vs baseline: 1.2432x; 1.2432x over previous
import functools

import jax
import jax.numpy as jnp
from jax import lax
from jax.experimental import pallas as pl
from jax.experimental.pallas import tpu as pltpu
from jax.experimental.pallas import tpu_sc as plsc

N_NODES = 10000
N_EDGES = 320000
D_FEAT = 128
N_HIDDEN = 96
HP = 128

_NC = 2
_NS = 16
_NW = _NC * _NS
_EPT = N_EDGES // _NW
_CH = 200
_NCHUNK = _EPT // _CH
_CHS = 80
_NCHUNKS = _EPT // _CHS
_ZB = 1000

_EDGE_BLK = 3200
_NODE_BLK = 2000


def _sc_mesh():
    return plsc.VectorSubcoreMesh(
        core_axis_name="c", subcore_axis_name="s",
        num_cores=_NC, num_subcores=_NS)


def _pad_cols(w):
    pad = [(0, 0)] * (w.ndim - 1) + [(0, HP - w.shape[-1])]
    return jnp.pad(w, pad)


def _gather_body(ept, a_hbm, b_hbm, src_hbm, dst_hbm, e_hbm,
                 idx_s, idx_d, rows, sem_a, sem_b, sem_o):
    nchunk = ept // _CH
    c = lax.axis_index("c")
    s = lax.axis_index("s")
    wid = s * _NC + c
    base0 = wid * ept

    pltpu.sync_copy(src_hbm.at[pl.ds(base0, ept)], idx_s)
    pltpu.sync_copy(dst_hbm.at[pl.ds(base0, ept)], idx_d)

    def issue_a(k, p):
        pltpu.async_copy(a_hbm.at[idx_s.at[pl.ds(k * _CH, _CH)]],
                         rows.at[p], sem_a.at[p])

    def issue_b(k, p):
        pltpu.make_async_copy(a_hbm.at[idx_s.at[pl.ds(k * _CH, _CH)]],
                              rows.at[p], sem_a.at[p]).wait()
        pltpu.async_copy(b_hbm.at[idx_d.at[pl.ds(k * _CH, _CH)]],
                         rows.at[p], sem_b.at[p], add=True)

    def issue_store(k, p):
        pltpu.make_async_copy(b_hbm.at[idx_d.at[pl.ds(k * _CH, _CH)]],
                              rows.at[p], sem_b.at[p]).wait()
        pltpu.async_copy(rows.at[p], e_hbm.at[pl.ds(base0 + k * _CH, _CH)],
                         sem_o.at[p])

    def wait_store(k, p):
        pltpu.make_async_copy(rows.at[p],
                              e_hbm.at[pl.ds(base0 + k * _CH, _CH)],
                              sem_o.at[p]).wait()

    def stage(i, carry):
        p0 = lax.rem(i, 3)
        p1 = lax.rem(i + 2, 3)
        p2 = lax.rem(i + 1, 3)

        @pl.when(i < nchunk)
        def _():
            @pl.when(i >= 3)
            def _():
                wait_store(i - 3, p0)
            issue_a(i, p0)

        @pl.when((i >= 1) & (i <= nchunk))
        def _():
            issue_b(i - 1, p1)

        @pl.when(i >= 2)
        def _():
            issue_store(i - 2, p2)
        return carry

    lax.fori_loop(0, nchunk + 2, stage, 0)
    for j in range(max(0, nchunk - 3), nchunk):
        wait_store(j, j % 3)


@functools.partial(jax.jit, static_argnums=(4,))
def _sc_gather(a, b, src, dst, nedges):
    ept = nedges // _NW
    f = pl.kernel(
        functools.partial(_gather_body, ept),
        out_type=jax.ShapeDtypeStruct((nedges, HP), jnp.float32),
        mesh=_sc_mesh(),
        scratch_types=[
            pltpu.VMEM((ept,), jnp.int32),
            pltpu.VMEM((ept,), jnp.int32),
            pltpu.VMEM((3, _CH, HP), jnp.float32),
            pltpu.SemaphoreType.DMA((3,)),
            pltpu.SemaphoreType.DMA((3,)),
            pltpu.SemaphoreType.DMA((3,)),
        ],
    )
    return f(a, b, src, dst)


def _scatter_body(ept, chs, m_hbm, dst_hbm, zeros_hbm, out_hbm,
                  idx, mbuf, acc, sem_m):
    nchunks = ept // chs
    c = lax.axis_index("c")
    s = lax.axis_index("s")
    wid = s * _NC + c
    base0 = wid * ept

    @pl.when(s < 10)
    def _zero():
        pltpu.sync_copy(zeros_hbm.at[pl.ds(s * _ZB, _ZB)],
                        acc.at[pl.ds(s * _ZB, _ZB)])
    pltpu.sync_copy(dst_hbm.at[pl.ds(base0, ept)], idx)
    plsc.subcore_barrier()

    def load(k, p):
        pltpu.async_copy(m_hbm.at[pl.ds(base0 + k * chs, chs)],
                         mbuf.at[p], sem_m.at[p])

    def add(k, p):
        pltpu.make_async_copy(m_hbm.at[pl.ds(base0 + k * chs, chs)],
                              mbuf.at[p], sem_m.at[p]).wait()
        pltpu.sync_copy(mbuf.at[p], acc.at[idx.at[pl.ds(k * chs, chs)]],
                        add=True)

    load(0, 0)

    def pair(i, carry):
        k0 = 2 * i
        k1 = k0 + 1
        load(k1, 1)
        add(k0, 0)

        @pl.when(k0 + 2 < nchunks)
        def _():
            load(k0 + 2, 0)
        add(k1, 1)
        return carry

    lax.fori_loop(0, nchunks // 2, pair, 0)
    if nchunks % 2:
        add(nchunks - 1, 0)
    plsc.subcore_barrier()

    @pl.when(s < 10)
    def _writeback():
        pltpu.sync_copy(acc.at[pl.ds(s * _ZB, _ZB)],
                        out_hbm.at[pl.ds(c * N_NODES + s * _ZB, _ZB)])


@functools.partial(jax.jit, static_argnums=(3, 4))
def _sc_scatter(m, dst3, zeros, nedges, chs):
    ept = nedges // _NW
    nchunks = ept // chs
    f = pl.kernel(
        functools.partial(_scatter_body, ept, chs),
        out_type=jax.ShapeDtypeStruct((2 * N_NODES, HP), jnp.float32),
        mesh=_sc_mesh(),
        scratch_types=[
            pltpu.VMEM((ept,), jnp.int32),
            pltpu.VMEM((2, chs, HP), jnp.float32),
            pltpu.VMEM_SHARED((N_NODES, HP), jnp.float32),
            pltpu.SemaphoreType.DMA((2,)),
        ],
    )
    return f(m, dst3, zeros)


def _dot(x, w):
    return jax.lax.dot_general(x, w, (((1,), (0,)), ((), ())),
                               preferred_element_type=jnp.float32)


def _pre_body(x_ref, w0, b0, w1, b1, w2, b2, w3, b3, wa, ba, wb,
              h_ref, a_ref, bo_ref):
    h = jnp.maximum(_dot(x_ref[...], w0[...]) + b0[...], 0.0)
    h = jnp.maximum(_dot(h, w1[...]) + b1[...], 0.0)
    h = jnp.maximum(_dot(h, w2[...]) + b2[...], 0.0)
    h = _dot(h, w3[...]) + b3[...]
    h_ref[...] = h
    a_ref[...] = _dot(h, wa[...]) + ba[...]
    bo_ref[...] = _dot(h, wb[...])


@jax.jit
def _tc_pre(x, pre, wa, ba, wb):
    nblk = N_NODES // _NODE_BLK
    row = lambda i: (i, 0)
    cst = lambda i: (0, 0)
    ws = []
    specs = [pl.BlockSpec((_NODE_BLK, D_FEAT), row)]
    for (w, b) in pre:
        ws += [w, b.reshape(1, -1)]
        specs += [pl.BlockSpec(w.shape, cst), pl.BlockSpec((1, w.shape[1]), cst)]
    ws += [wa, ba.reshape(1, -1), wb]
    specs += [pl.BlockSpec(wa.shape, cst), pl.BlockSpec((1, HP), cst),
              pl.BlockSpec(wb.shape, cst)]
    return pl.pallas_call(
        _pre_body,
        grid=(nblk,),
        in_specs=specs,
        out_specs=[pl.BlockSpec((_NODE_BLK, N_HIDDEN), row),
                   pl.BlockSpec((_NODE_BLK, HP), row),
                   pl.BlockSpec((_NODE_BLK, HP), row)],
        out_shape=[jax.ShapeDtypeStruct((N_NODES, N_HIDDEN), jnp.float32),
                   jax.ShapeDtypeStruct((N_NODES, HP), jnp.float32),
                   jax.ShapeDtypeStruct((N_NODES, HP), jnp.float32)],
    )(x, *ws)


def _edge_body(e_ref, w2, b2, w3, b3, w4, b4, m_ref):
    e = jnp.maximum(e_ref[...], 0.0)
    e = jnp.maximum(_dot(e, w2[...]) + b2[...], 0.0)
    e = jnp.maximum(_dot(e, w3[...]) + b3[...], 0.0)
    m_ref[...] = _dot(e, w4[...]) + b4[...]


@jax.jit
def _tc_edge(e1, w2p, b2, w3, b3, w4p, b4p):
    nblk = N_EDGES // _EDGE_BLK
    row = lambda i: (i, 0)
    cst = lambda i: (0, 0)
    ws = [w2p, b2.reshape(1, -1), w3, b3.reshape(1, -1),
          w4p, b4p.reshape(1, -1)]
    specs = [pl.BlockSpec((_EDGE_BLK, HP), row)]
    for w in ws:
        specs.append(pl.BlockSpec(w.shape, cst))
    return pl.pallas_call(
        _edge_body,
        grid=(nblk,),
        in_specs=specs,
        out_specs=pl.BlockSpec((_EDGE_BLK, HP), row),
        out_shape=jax.ShapeDtypeStruct((N_EDGES, HP), jnp.float32),
    )(e1, *ws)


def _node_call(h, parts, node, heads, out_dims, final):
    p0 = parts[:N_NODES]
    p1 = parts[N_NODES:]
    nblk = N_NODES // _NODE_BLK
    row = lambda i: (i, 0)
    cst = lambda i: (0, 0)
    (wn1, bn1), n2, n3, n4 = node
    wna = wn1[:N_HIDDEN]
    wnb = jnp.pad(wn1[N_HIDDEN:], ((0, HP - N_HIDDEN), (0, 0)))
    ws = [wna, wnb, bn1.reshape(1, -1)]
    specs = [pl.BlockSpec((_NODE_BLK, N_HIDDEN), row)]
    specs += [pl.BlockSpec((_NODE_BLK, HP), row)] * 2
    specs += [pl.BlockSpec(wna.shape, cst), pl.BlockSpec(wnb.shape, cst),
              pl.BlockSpec((1, N_HIDDEN), cst)]
    for (w, b) in (n2, n3, n4):
        ws += [w, b.reshape(1, -1)]
        specs += [pl.BlockSpec(w.shape, cst), pl.BlockSpec((1, w.shape[1]), cst)]
    for hmat in heads:
        shp = hmat.shape if hmat.ndim == 2 else (1, hmat.shape[-1])
        ws.append(hmat.reshape(shp))
        specs.append(pl.BlockSpec(shp, cst))

    def body(h_ref, p0_ref, p1_ref, rwna, rwnb, rbn1,
             rw2, rb2, rw3, rb3, rw4, rb4, *rest):
        agg = p0_ref[...] + p1_ref[...]
        e = jnp.maximum(_dot(h_ref[...], rwna[...]) + _dot(agg, rwnb[...])
                        + rbn1[...], 0.0)
        e = jnp.maximum(_dot(e, rw2[...]) + rb2[...], 0.0)
        e = jnp.maximum(_dot(e, rw3[...]) + rb3[...], 0.0)
        hn = _dot(e, rw4[...]) + rb4[...]
        if final:
            rwo, rbo, out_ref = rest
            out_ref[...] = _dot(hn, rwo[...]) + rbo[...]
        else:
            rwa, rba, rwb, h_out, a_out, b_out = rest
            h_out[...] = hn
            a_out[...] = _dot(hn, rwa[...]) + rba[...]
            b_out[...] = _dot(hn, rwb[...])

    return pl.pallas_call(
        body,
        grid=(nblk,),
        in_specs=specs,
        out_specs=[pl.BlockSpec((_NODE_BLK, d), row) for d, _ in out_dims],
        out_shape=[jax.ShapeDtypeStruct((N_NODES, d), t)
                   for d, t in out_dims],
    )(h, p0, p1, *ws)


@jax.jit
def _tc_node_mid(h, parts, node, wa, ba, wb):
    return _node_call(h, parts, node, [wa, ba, wb],
                      [(N_HIDDEN, jnp.float32), (HP, jnp.float32),
                       (HP, jnp.float32)], final=False)


@jax.jit
def _tc_node_final(h, parts, node, wo, bo):
    return _node_call(h, parts, node, [wo, bo], [(D_FEAT, jnp.float32)],
                      final=True)


def kernel(x, params, edge_index):
    src = edge_index[0].astype(jnp.int32)
    dst = edge_index[1].astype(jnp.int32)
    msg = params['msg']
    wm1, bm1 = msg[0]
    wma = _pad_cols(wm1[:N_HIDDEN])
    wmb = _pad_cols(wm1[N_HIDDEN:])
    bm1p = _pad_cols(bm1)
    (w2, b2), (w3, b3), (w4, b4) = msg[1], msg[2], msg[3]
    w2p = jnp.pad(w2, ((0, HP - N_HIDDEN), (0, 0)))
    w4p = _pad_cols(w4)
    b4p = _pad_cols(b4)
    wo, bo = params['out']

    h, a, b = _tc_pre(x, params['pre'], wma, bm1p, wmb)
    zeros = jnp.zeros((N_NODES, HP), jnp.float32)

    out = None
    for step in range(2):
        e1 = _sc_gather(a, b, src, dst, N_EDGES)
        m = _tc_edge(e1, w2p, b2, w3, b3, w4p, b4p)
        parts = _sc_scatter(m, dst, zeros, N_EDGES, _CHS)
        if step == 0:
            h, a, b = _tc_node_mid(h, parts, params['node'], wma, bm1p, wmb)
        else:
            (out,) = _tc_node_final(h, parts, params['node'], wo, bo)
    return out


# speedup vs baseline: 1.2872x; 1.0354x over previous
import functools

import jax
import jax.numpy as jnp
from jax import lax
from jax.experimental import pallas as pl
from jax.experimental.pallas import tpu as pltpu
from jax.experimental.pallas import tpu_sc as plsc

N_NODES = 10000
N_EDGES = 320000
D_FEAT = 128
N_HIDDEN = 96
HP = 128

_NC = 2
_NS = 16
_NW = _NC * _NS
_EPT = N_EDGES // _NW
_CH = 200
_NCHUNK = _EPT // _CH
_CHS = 80
_NCHUNKS = _EPT // _CHS
_ZB = 1000

_EDGE_BLK = 3200
_NODE_BLK = 2000


def _sc_mesh():
    return plsc.VectorSubcoreMesh(
        core_axis_name="c", subcore_axis_name="s",
        num_cores=_NC, num_subcores=_NS)


def _pad_cols(w):
    pad = [(0, 0)] * (w.ndim - 1) + [(0, HP - w.shape[-1])]
    return jnp.pad(w, pad)


def _gather_body(ept, a_hbm, b_hbm, src_hbm, dst_hbm, e_hbm,
                 idx_s, idx_d, rows, sem_a, sem_b, sem_o):
    nchunk = ept // _CH
    c = lax.axis_index("c")
    s = lax.axis_index("s")
    wid = s * _NC + c
    base0 = wid * ept

    pltpu.sync_copy(src_hbm.at[pl.ds(base0, ept)], idx_s)
    pltpu.sync_copy(dst_hbm.at[pl.ds(base0, ept)], idx_d)

    def issue_a(k, p):
        pltpu.async_copy(a_hbm.at[idx_s.at[pl.ds(k * _CH, _CH)]],
                         rows.at[p], sem_a.at[p])

    def issue_b(k, p):
        pltpu.make_async_copy(a_hbm.at[idx_s.at[pl.ds(k * _CH, _CH)]],
                              rows.at[p], sem_a.at[p]).wait()
        pltpu.async_copy(b_hbm.at[idx_d.at[pl.ds(k * _CH, _CH)]],
                         rows.at[p], sem_b.at[p], add=True)

    def issue_store(k, p):
        pltpu.make_async_copy(b_hbm.at[idx_d.at[pl.ds(k * _CH, _CH)]],
                              rows.at[p], sem_b.at[p]).wait()
        pltpu.async_copy(rows.at[p], e_hbm.at[pl.ds(base0 + k * _CH, _CH)],
                         sem_o.at[p])

    def wait_store(k, p):
        pltpu.make_async_copy(rows.at[p],
                              e_hbm.at[pl.ds(base0 + k * _CH, _CH)],
                              sem_o.at[p]).wait()

    def stage(i, carry):
        p0 = lax.rem(i, 4)
        p1 = lax.rem(i + 3, 4)
        p2 = lax.rem(i + 2, 4)

        @pl.when(i < nchunk)
        def _():
            @pl.when(i >= 4)
            def _():
                wait_store(i - 4, p0)
            issue_a(i, p0)

        @pl.when((i >= 1) & (i <= nchunk))
        def _():
            issue_b(i - 1, p1)

        @pl.when(i >= 2)
        def _():
            issue_store(i - 2, p2)
        return carry

    lax.fori_loop(0, nchunk + 2, stage, 0)
    for j in range(max(0, nchunk - 4), nchunk):
        wait_store(j, j % 4)


@functools.partial(jax.jit, static_argnums=(4,))
def _sc_gather(a, b, src, dst, nedges):
    ept = nedges // _NW
    f = pl.kernel(
        functools.partial(_gather_body, ept),
        out_type=jax.ShapeDtypeStruct((nedges, HP), jnp.float32),
        mesh=_sc_mesh(),
        scratch_types=[
            pltpu.VMEM((ept,), jnp.int32),
            pltpu.VMEM((ept,), jnp.int32),
            pltpu.VMEM((4, _CH, HP), jnp.float32),
            pltpu.SemaphoreType.DMA((4,)),
            pltpu.SemaphoreType.DMA((4,)),
            pltpu.SemaphoreType.DMA((4,)),
        ],
    )
    return f(a, b, src, dst)


def _scatter_body(ept, chs, m_hbm, dst_hbm, zeros_hbm, out_hbm,
                  idx, mbuf, acc, sem_m, sem_add):
    nchunks = ept // chs
    c = lax.axis_index("c")
    s = lax.axis_index("s")
    wid = s * _NC + c
    base0 = wid * ept

    @pl.when(s < 10)
    def _zero():
        pltpu.sync_copy(zeros_hbm.at[pl.ds(s * _ZB, _ZB)],
                        acc.at[pl.ds(s * _ZB, _ZB)])
    pltpu.sync_copy(dst_hbm.at[pl.ds(base0, ept)], idx)
    plsc.subcore_barrier()

    def load(k, p):
        pltpu.async_copy(m_hbm.at[pl.ds(base0 + k * chs, chs)],
                         mbuf.at[p], sem_m.at[p])

    def add(k, p):
        pltpu.make_async_copy(m_hbm.at[pl.ds(base0 + k * chs, chs)],
                              mbuf.at[p], sem_m.at[p]).wait()
        pltpu.async_copy(mbuf.at[p], acc.at[idx.at[pl.ds(k * chs, chs)]],
                         sem_add.at[p], add=True)

    def wait_add(k, p):
        pltpu.make_async_copy(mbuf.at[p],
                              acc.at[idx.at[pl.ds(k * chs, chs)]],
                              sem_add.at[p]).wait()

    def stage(i, carry):
        p0 = lax.rem(i, 3)
        p1 = lax.rem(i + 2, 3)

        @pl.when(i < nchunks)
        def _():
            @pl.when(i >= 3)
            def _():
                wait_add(i - 3, p0)
            load(i, p0)

        @pl.when(i >= 1)
        def _():
            add(i - 1, p1)
        return carry

    lax.fori_loop(0, nchunks + 1, stage, 0)
    for j in range(max(0, nchunks - 3), nchunks):
        wait_add(j, j % 3)
    plsc.subcore_barrier()

    @pl.when(s < 10)
    def _writeback():
        pltpu.sync_copy(acc.at[pl.ds(s * _ZB, _ZB)],
                        out_hbm.at[pl.ds(c * N_NODES + s * _ZB, _ZB)])


@functools.partial(jax.jit, static_argnums=(3, 4))
def _sc_scatter(m, dst3, zeros, nedges, chs):
    ept = nedges // _NW
    nchunks = ept // chs
    f = pl.kernel(
        functools.partial(_scatter_body, ept, chs),
        out_type=jax.ShapeDtypeStruct((2 * N_NODES, HP), jnp.float32),
        mesh=_sc_mesh(),
        scratch_types=[
            pltpu.VMEM((ept,), jnp.int32),
            pltpu.VMEM((3, chs, HP), jnp.float32),
            pltpu.VMEM_SHARED((N_NODES, HP), jnp.float32),
            pltpu.SemaphoreType.DMA((3,)),
            pltpu.SemaphoreType.DMA((3,)),
        ],
    )
    return f(m, dst3, zeros)


def _dot(x, w):
    return jax.lax.dot_general(x, w, (((1,), (0,)), ((), ())),
                               preferred_element_type=jnp.float32)


def _pre_body(x_ref, w0, b0, w1, b1, w2, b2, w3, b3, wa, ba, wb,
              h_ref, a_ref, bo_ref):
    h = jnp.maximum(_dot(x_ref[...], w0[...]) + b0[...], 0.0)
    h = jnp.maximum(_dot(h, w1[...]) + b1[...], 0.0)
    h = jnp.maximum(_dot(h, w2[...]) + b2[...], 0.0)
    h = _dot(h, w3[...]) + b3[...]
    h_ref[...] = h
    a_ref[...] = _dot(h, wa[...]) + ba[...]
    bo_ref[...] = _dot(h, wb[...])


@jax.jit
def _tc_pre(x, pre, wa, ba, wb):
    nblk = N_NODES // _NODE_BLK
    row = lambda i: (i, 0)
    cst = lambda i: (0, 0)
    ws = []
    specs = [pl.BlockSpec((_NODE_BLK, D_FEAT), row)]
    for (w, b) in pre:
        ws += [w, b.reshape(1, -1)]
        specs += [pl.BlockSpec(w.shape, cst), pl.BlockSpec((1, w.shape[1]), cst)]
    ws += [wa, ba.reshape(1, -1), wb]
    specs += [pl.BlockSpec(wa.shape, cst), pl.BlockSpec((1, HP), cst),
              pl.BlockSpec(wb.shape, cst)]
    return pl.pallas_call(
        _pre_body,
        grid=(nblk,),
        in_specs=specs,
        out_specs=[pl.BlockSpec((_NODE_BLK, N_HIDDEN), row),
                   pl.BlockSpec((_NODE_BLK, HP), row),
                   pl.BlockSpec((_NODE_BLK, HP), row)],
        out_shape=[jax.ShapeDtypeStruct((N_NODES, N_HIDDEN), jnp.float32),
                   jax.ShapeDtypeStruct((N_NODES, HP), jnp.float32),
                   jax.ShapeDtypeStruct((N_NODES, HP), jnp.float32)],
    )(x, *ws)


def _edge_body(e_ref, w2, b2, w3, b3, w4, b4, m_ref):
    e = jnp.maximum(e_ref[...], 0.0)
    e = jnp.maximum(_dot(e, w2[...]) + b2[...], 0.0)
    e = jnp.maximum(_dot(e, w3[...]) + b3[...], 0.0)
    m_ref[...] = _dot(e, w4[...]) + b4[...]


@jax.jit
def _tc_edge(e1, w2p, b2, w3, b3, w4p, b4p):
    nblk = N_EDGES // _EDGE_BLK
    row = lambda i: (i, 0)
    cst = lambda i: (0, 0)
    ws = [w2p, b2.reshape(1, -1), w3, b3.reshape(1, -1),
          w4p, b4p.reshape(1, -1)]
    specs = [pl.BlockSpec((_EDGE_BLK, HP), row)]
    for w in ws:
        specs.append(pl.BlockSpec(w.shape, cst))
    return pl.pallas_call(
        _edge_body,
        grid=(nblk,),
        in_specs=specs,
        out_specs=pl.BlockSpec((_EDGE_BLK, HP), row),
        out_shape=jax.ShapeDtypeStruct((N_EDGES, HP), jnp.float32),
    )(e1, *ws)


def _node_call(h, parts, node, heads, out_dims, final):
    p0 = parts[:N_NODES]
    p1 = parts[N_NODES:]
    nblk = N_NODES // _NODE_BLK
    row = lambda i: (i, 0)
    cst = lambda i: (0, 0)
    (wn1, bn1), n2, n3, n4 = node
    wna = wn1[:N_HIDDEN]
    wnb = jnp.pad(wn1[N_HIDDEN:], ((0, HP - N_HIDDEN), (0, 0)))
    ws = [wna, wnb, bn1.reshape(1, -1)]
    specs = [pl.BlockSpec((_NODE_BLK, N_HIDDEN), row)]
    specs += [pl.BlockSpec((_NODE_BLK, HP), row)] * 2
    specs += [pl.BlockSpec(wna.shape, cst), pl.BlockSpec(wnb.shape, cst),
              pl.BlockSpec((1, N_HIDDEN), cst)]
    for (w, b) in (n2, n3, n4):
        ws += [w, b.reshape(1, -1)]
        specs += [pl.BlockSpec(w.shape, cst), pl.BlockSpec((1, w.shape[1]), cst)]
    for hmat in heads:
        shp = hmat.shape if hmat.ndim == 2 else (1, hmat.shape[-1])
        ws.append(hmat.reshape(shp))
        specs.append(pl.BlockSpec(shp, cst))

    def body(h_ref, p0_ref, p1_ref, rwna, rwnb, rbn1,
             rw2, rb2, rw3, rb3, rw4, rb4, *rest):
        agg = p0_ref[...] + p1_ref[...]
        e = jnp.maximum(_dot(h_ref[...], rwna[...]) + _dot(agg, rwnb[...])
                        + rbn1[...], 0.0)
        e = jnp.maximum(_dot(e, rw2[...]) + rb2[...], 0.0)
        e = jnp.maximum(_dot(e, rw3[...]) + rb3[...], 0.0)
        hn = _dot(e, rw4[...]) + rb4[...]
        if final:
            rwo, rbo, out_ref = rest
            out_ref[...] = _dot(hn, rwo[...]) + rbo[...]
        else:
            rwa, rba, rwb, h_out, a_out, b_out = rest
            h_out[...] = hn
            a_out[...] = _dot(hn, rwa[...]) + rba[...]
            b_out[...] = _dot(hn, rwb[...])

    return pl.pallas_call(
        body,
        grid=(nblk,),
        in_specs=specs,
        out_specs=[pl.BlockSpec((_NODE_BLK, d), row) for d, _ in out_dims],
        out_shape=[jax.ShapeDtypeStruct((N_NODES, d), t)
                   for d, t in out_dims],
    )(h, p0, p1, *ws)


@jax.jit
def _tc_node_mid(h, parts, node, wa, ba, wb):
    return _node_call(h, parts, node, [wa, ba, wb],
                      [(N_HIDDEN, jnp.float32), (HP, jnp.float32),
                       (HP, jnp.float32)], final=False)


@jax.jit
def _tc_node_final(h, parts, node, wo, bo):
    return _node_call(h, parts, node, [wo, bo], [(D_FEAT, jnp.float32)],
                      final=True)


def kernel(x, params, edge_index):
    src = edge_index[0].astype(jnp.int32)
    dst = edge_index[1].astype(jnp.int32)
    msg = params['msg']
    wm1, bm1 = msg[0]
    wma = _pad_cols(wm1[:N_HIDDEN])
    wmb = _pad_cols(wm1[N_HIDDEN:])
    bm1p = _pad_cols(bm1)
    (w2, b2), (w3, b3), (w4, b4) = msg[1], msg[2], msg[3]
    w2p = jnp.pad(w2, ((0, HP - N_HIDDEN), (0, 0)))
    w4p = _pad_cols(w4)
    b4p = _pad_cols(b4)
    wo, bo = params['out']

    h, a, b = _tc_pre(x, params['pre'], wma, bm1p, wmb)
    zeros = jnp.zeros((N_NODES, HP), jnp.float32)

    out = None
    for step in range(2):
        e1 = _sc_gather(a, b, src, dst, N_EDGES)
        m = _tc_edge(e1, w2p, b2, w3, b3, w4p, b4p)
        parts = _sc_scatter(m, dst, zeros, N_EDGES, _CHS)
        if step == 0:
            h, a, b = _tc_node_mid(h, parts, params['node'], wma, bm1p, wmb)
        else:
            (out,) = _tc_node_final(h, parts, params['node'], wo, bo)
    return out
